# Initial kernel scaffold; baseline (speedup 1.0000x reference)
import jax
import jax.numpy as jnp
from jax.experimental import pallas as pl


def kernel(pred, target):
    raise NotImplementedError("write your pallas kernel here")


# speedup vs baseline: 1.1215x; 1.1215x over previous
import jax
import jax.numpy as jnp
from jax.experimental import pallas as pl
from jax.experimental.pallas import tpu as pltpu

_MU = 0.02
_NBINS = 10
_ALPHA = 0.75
_N = 8388608
_COLS = 128
_ROWS = _N // _COLS
_BLK = 4096
_GRID = _ROWS // _BLK


def _ghmr_body(p_ref, t_ref, out_ref, acc_ref):
    step = pl.program_id(0)

    @pl.when(step == 0)
    def _init():
        for k in range(2 * _NBINS + 1):
            acc_ref[k] = jnp.float32(0.0)

    p = p_ref[...]
    t = t_ref[...]
    d = jnp.abs(p - t)
    loss = jnp.where(d < _MU, (0.5 / _MU) * d * d, d - 0.5 * _MU)
    g = jnp.abs(jnp.tanh(p) - jnp.tanh(t))
    b = (g * _NBINS).astype(jnp.int32)
    bl = jnp.minimum(b, _NBINS - 1)

    for k in range(_NBINS):
        m = bl == k
        mf = m.astype(jnp.float32)
        acc_ref[k] += jnp.sum(mf)
        acc_ref[_NBINS + k] += jnp.sum(jnp.where(m, loss, 0.0))
    acc_ref[2 * _NBINS] += jnp.sum((b >= _NBINS).astype(jnp.float32))

    @pl.when(step == _GRID - 1)
    def _finish():
        total = jnp.float32(0.0)
        for k in range(_NBINS):
            cnt = acc_ref[k]
            if k == _NBINS - 1:
                cnt = cnt - acc_ref[2 * _NBINS]
            cnt = jnp.maximum(cnt, 1.0)
            w = jnp.exp(-_ALPHA * jnp.log(cnt))
            total = total + w * acc_ref[_NBINS + k]
        out_ref[0] = total / _N


def kernel(pred, target):
    p2 = pred.reshape(_ROWS, _COLS)
    t2 = target.reshape(_ROWS, _COLS)
    out = pl.pallas_call(
        _ghmr_body,
        grid=(_GRID,),
        in_specs=[
            pl.BlockSpec((_BLK, _COLS), lambda i: (i, 0)),
            pl.BlockSpec((_BLK, _COLS), lambda i: (i, 0)),
        ],
        out_specs=pl.BlockSpec(memory_space=pltpu.SMEM),
        out_shape=jax.ShapeDtypeStruct((1,), jnp.float32),
        scratch_shapes=[pltpu.SMEM((2 * _NBINS + 1,), jnp.float32)],
    )(p2, t2)
    return out[0]
